# Initial kernel scaffold; baseline (speedup 1.0000x reference)
import jax
import jax.numpy as jnp
from jax.experimental import pallas as pl


def kernel(x, w_gate, fc1_w, fc1_b, fc2_w, fc2_b):
    raise NotImplementedError("write your pallas kernel here")


# speedup vs baseline: 2.5416x; 2.5416x over previous
import jax
import jax.numpy as jnp
from jax.experimental import pallas as pl
from jax.experimental.pallas import tpu as pltpu

_N, _D, _E, _H = 2048, 768, 8, 3072
_CH = 256


def _gate_kernel(x_ref, wg_ref, gates_ref, loss_ref):
    x = x_ref[...]
    wg = wg_ref[...]
    logits = jax.lax.dot_general(
        x, wg, (((1,), (0,)), ((), ())),
        preferred_element_type=jnp.float32)
    iota = jax.lax.broadcasted_iota(jnp.int32, (_N, _E), 1)
    m1 = jnp.max(logits, axis=1, keepdims=True)
    i1 = jnp.min(jnp.where(logits == m1, iota, _E), axis=1, keepdims=True)
    masked = jnp.where(iota == i1, -jnp.inf, logits)
    m2 = jnp.max(masked, axis=1, keepdims=True)
    i2 = jnp.min(jnp.where(masked == m2, iota, _E), axis=1, keepdims=True)
    ew = jnp.exp(m2 - m1)
    denom = 1.0 + ew
    w1 = 1.0 / denom
    w2 = ew / denom
    gates = jnp.where(iota == i1, w1, 0.0) + jnp.where(iota == i2, w2, 0.0)
    gates_ref[...] = gates
    load = jnp.sum((gates > 0.0).astype(jnp.float32), axis=0)
    mean = jnp.mean(load)
    var = jnp.sum((load - mean) ** 2) / (_E - 1)
    loss_ref[...] = jnp.full((1, 1), var / (mean * mean + 1e-6), jnp.float32)


def _ffn_kernel(x_ref, gates_ref, w1_ref, b1_ref, w2_ref, b2_ref, out_ref):
    e = pl.program_id(0)
    iota = jax.lax.broadcasted_iota(jnp.int32, (_N, _E), 1)
    g = jnp.sum(jnp.where(iota == e, gates_ref[...], 0.0), axis=1,
                keepdims=True)
    w1 = w1_ref[0]
    w2 = w2_ref[0]
    b1 = b1_ref[0]
    b2 = b2_ref[0]

    @pl.when(e == 0)
    def _init():
        out_ref[...] = jnp.zeros_like(out_ref)

    xb = x_ref[...].astype(jnp.bfloat16)
    for c in range(_N // _CH):
        lo, hi = c * _CH, (c + 1) * _CH
        xc = xb[lo:hi, :]
        h = jnp.dot(xc, w1, preferred_element_type=jnp.float32) + b1
        h = 0.5 * h * (1.0 + jax.lax.erf(h * 0.7071067811865476))
        y = jnp.dot(h.astype(jnp.bfloat16), w2,
                    preferred_element_type=jnp.float32) + b2
        out_ref[lo:hi, :] += y * g[lo:hi, :]


def kernel(x, w_gate, fc1_w, fc1_b, fc2_w, fc2_b):
    x2 = x.reshape(_N, _D)
    gates, loss = pl.pallas_call(
        _gate_kernel,
        out_shape=(
            jax.ShapeDtypeStruct((_N, _E), jnp.float32),
            jax.ShapeDtypeStruct((1, 1), jnp.float32),
        ),
    )(x2, w_gate)

    w1b = fc1_w.astype(jnp.bfloat16)
    w2b = fc2_w.astype(jnp.bfloat16)
    b1r = fc1_b.reshape(_E, 1, _H)
    b2r = fc2_b.reshape(_E, 1, _D)

    out = pl.pallas_call(
        _ffn_kernel,
        grid=(_E,),
        in_specs=[
            pl.BlockSpec((_N, _D), lambda e: (0, 0)),
            pl.BlockSpec((_N, _E), lambda e: (0, 0)),
            pl.BlockSpec((1, _D, _H), lambda e: (e, 0, 0)),
            pl.BlockSpec((1, 1, _H), lambda e: (e, 0, 0)),
            pl.BlockSpec((1, _H, _D), lambda e: (e, 0, 0)),
            pl.BlockSpec((1, 1, _D), lambda e: (e, 0, 0)),
        ],
        out_specs=pl.BlockSpec((_N, _D), lambda e: (0, 0)),
        out_shape=jax.ShapeDtypeStruct((_N, _D), jnp.float32),
        compiler_params=pltpu.CompilerParams(
            dimension_semantics=("arbitrary",)),
    )(x2, gates, w1b, b1r, w2b, b2r)

    return out.reshape(x.shape), loss.reshape(())


# speedup vs baseline: 2.9430x; 1.1579x over previous
import jax
import jax.numpy as jnp
from jax import lax
from jax.experimental import pallas as pl
from jax.experimental.pallas import tpu as pltpu
from jax.experimental.pallas import tpu_sc as plsc

_N, _D, _E, _H = 2048, 768, 8, 3072
_K = 2
_P = _N * _K
_TB = 256
_CAP = _P + _E * _TB
_NBLK = _CAP // _TB
_NW = 32
_PPW = _P // _NW
_TPW = _N // _NW


def _gate_kernel(x_ref, wg_ref, d0_ref, d1_ref, w1_ref, w2_ref, binfo_ref,
                 loss_ref):
    x = x_ref[...]
    wg = wg_ref[...]
    logits = jax.lax.dot_general(
        x, wg, (((1,), (0,)), ((), ())),
        preferred_element_type=jnp.float32)
    iota = jax.lax.broadcasted_iota(jnp.int32, (_N, _E), 1)
    m1 = jnp.max(logits, axis=1, keepdims=True)
    i1 = jnp.min(jnp.where(logits == m1, iota, _E), axis=1, keepdims=True)
    masked = jnp.where(iota == i1, -jnp.inf, logits)
    m2 = jnp.max(masked, axis=1, keepdims=True)
    i2 = jnp.min(jnp.where(masked == m2, iota, _E), axis=1, keepdims=True)
    ew = jnp.exp(m2 - m1)
    denom = 1.0 + ew
    w1_ref[...] = 1.0 / denom
    w2_ref[...] = ew / denom

    gates = (jnp.where(iota == i1, 1.0 / denom, 0.0)
             + jnp.where(iota == i2, ew / denom, 0.0))
    load = jnp.sum((gates > 0.0).astype(jnp.float32), axis=0)
    mean = jnp.mean(load)
    var = jnp.sum((load - mean) ** 2) / (_E - 1)
    loss_ref[...] = jnp.full((1, 1), var / (mean * mean + 1e-6), jnp.float32)

    mask1 = jnp.where(iota == i1, 1.0, 0.0)
    mask2 = jnp.where(iota == i2, 1.0, 0.0)
    m12 = jnp.concatenate([mask1, mask2], axis=1)
    ir = jax.lax.broadcasted_iota(jnp.int32, (_N, _N), 0)
    ic = jax.lax.broadcasted_iota(jnp.int32, (_N, _N), 1)
    tri = jnp.where(ir >= ic, 1.0, 0.0)
    cs = jax.lax.dot_general(
        tri, m12, (((1,), (0,)), ((), ())),
        preferred_element_type=jnp.float32,
        precision=jax.lax.Precision.HIGHEST)
    cs1 = cs[:, 0:_E]
    cs2 = cs[:, _E:2 * _E]
    tot1 = cs1[_N - 1:_N, :]
    tot2 = cs2[_N - 1:_N, :]
    counts = tot1 + tot2
    padded = jnp.ceil(counts / _TB) * _TB
    ie_r = jax.lax.broadcasted_iota(jnp.int32, (_E, _E), 0)
    ie_c = jax.lax.broadcasted_iota(jnp.int32, (_E, _E), 1)
    padb = jnp.broadcast_to(padded, (_E, _E))
    offp = jnp.sum(jnp.where(ie_c < ie_r, padb, 0.0), axis=1,
                   keepdims=False).reshape(1, _E)

    rank1 = jnp.sum(jnp.where(iota == i1, cs1, 0.0), axis=1,
                    keepdims=True) - 1.0
    rank2 = (jnp.sum(jnp.where(iota == i2, cs2 + tot1, 0.0), axis=1,
                     keepdims=True) - 1.0)
    base1 = jnp.sum(jnp.where(iota == i1, offp, 0.0), axis=1, keepdims=True)
    base2 = jnp.sum(jnp.where(iota == i2, offp, 0.0), axis=1, keepdims=True)
    d0_ref[...] = (base1 + rank1).astype(jnp.int32)
    d1_ref[...] = (base2 + rank2).astype(jnp.int32)

    ib = jax.lax.broadcasted_iota(jnp.int32, (32, _E), 0)
    offp32 = jnp.broadcast_to(offp, (32, _E))
    bexp = jnp.sum(
        jnp.where((ib * _TB).astype(jnp.float32) >= offp32, 1.0, 0.0),
        axis=1) - 1.0
    nblk = jnp.sum(padded) / _TB
    i32v = jax.lax.broadcasted_iota(jnp.int32, (32,), 0)
    binfo_ref[...] = jnp.where(i32v == 31, nblk, bexp).astype(jnp.int32)


def _dispatch_kernel(dest_hbm, x_hbm, xs_hbm, dest_v, idx_v, rows_v, sem):
    c = lax.axis_index("c")
    s = lax.axis_index("s")
    wid = s * 2 + c
    my_start = wid * _PPW
    t0 = my_start - (my_start // _N) * _N
    pltpu.sync_copy(dest_hbm.at[pl.ds(my_start, _PPW)], dest_v)
    for cc in range(4):
        pltpu.sync_copy(x_hbm.at[pl.ds(t0 + cc * 32, 32)], rows_v)
        for q in range(2):
            idx_v[(q * 16):(q * 16 + 16)] = \
                dest_v[(cc * 32 + q * 16):(cc * 32 + q * 16 + 16)]
        pltpu.async_copy(rows_v, xs_hbm.at[idx_v], sem).wait()


def _ffn_kernel(binfo_ref, xs_ref, w1_ref, b1_ref, w2_ref, b2_ref, ys_ref):
    b = pl.program_id(0)
    nb = binfo_ref[31]

    @pl.when(b < nb)
    def _():
        xc = xs_ref[...].astype(jnp.bfloat16)
        h = jnp.dot(xc, w1_ref[0], preferred_element_type=jnp.float32)
        h = h + b1_ref[0]
        h = 0.5 * h * (1.0 + jax.lax.erf(h * 0.7071067811865476))
        y = jnp.dot(h.astype(jnp.bfloat16), w2_ref[0],
                    preferred_element_type=jnp.float32)
        ys_ref[...] = y + b2_ref[0]


def _gather_kernel(d0_hbm, d1_hbm, ys_hbm, y0_hbm, y1_hbm,
                   d0_v, d1_v, idx_v, rows_v, sem):
    c = lax.axis_index("c")
    s = lax.axis_index("s")
    wid = s * 2 + c
    tbase = wid * _TPW
    pltpu.sync_copy(d0_hbm.at[pl.ds(tbase, _TPW)], d0_v)
    pltpu.sync_copy(d1_hbm.at[pl.ds(tbase, _TPW)], d1_v)
    for cc in range(_TPW // 32):
        for q in range(2):
            idx_v[(q * 16):(q * 16 + 16)] = \
                d0_v[(cc * 32 + q * 16):(cc * 32 + q * 16 + 16)]
        pltpu.async_copy(ys_hbm.at[idx_v], rows_v, sem).wait()
        pltpu.sync_copy(rows_v, y0_hbm.at[pl.ds(tbase + cc * 32, 32)])
        for q in range(2):
            idx_v[(q * 16):(q * 16 + 16)] = \
                d1_v[(cc * 32 + q * 16):(cc * 32 + q * 16 + 16)]
        pltpu.async_copy(ys_hbm.at[idx_v], rows_v, sem).wait()
        pltpu.sync_copy(rows_v, y1_hbm.at[pl.ds(tbase + cc * 32, 32)])


def _mix_kernel(y0_ref, y1_ref, w1_ref, w2_ref, out_ref):
    out_ref[...] = y0_ref[...] * w1_ref[...] + y1_ref[...] * w2_ref[...]


def kernel(x, w_gate, fc1_w, fc1_b, fc2_w, fc2_b):
    x2 = x.reshape(_N, _D)
    d0, d1, w1c, w2c, binfo, loss = pl.pallas_call(
        _gate_kernel,
        out_shape=(
            jax.ShapeDtypeStruct((_N, 1), jnp.int32),
            jax.ShapeDtypeStruct((_N, 1), jnp.int32),
            jax.ShapeDtypeStruct((_N, 1), jnp.float32),
            jax.ShapeDtypeStruct((_N, 1), jnp.float32),
            jax.ShapeDtypeStruct((32,), jnp.int32),
            jax.ShapeDtypeStruct((1, 1), jnp.float32),
        ),
    )(x2, w_gate)

    dest = jnp.concatenate([d0.reshape(-1), d1.reshape(-1)])

    mesh = plsc.VectorSubcoreMesh(core_axis_name="c", subcore_axis_name="s")

    dispatch = pl.kernel(
        _dispatch_kernel,
        mesh=mesh,
        out_type=jax.ShapeDtypeStruct((_CAP, _D), jnp.float32),
        scratch_types=[
            pltpu.VMEM((_PPW,), jnp.int32),
            pltpu.VMEM((32,), jnp.int32),
            pltpu.VMEM((32, _D), jnp.float32),
            pltpu.SemaphoreType.DMA,
        ],
    )
    xs = dispatch(dest, x2)

    w1b = fc1_w.astype(jnp.bfloat16)
    w2b = fc2_w.astype(jnp.bfloat16)
    b1r = fc1_b.reshape(_E, 1, _H)
    b2r = fc2_b.reshape(_E, 1, _D)
    grid_spec = pltpu.PrefetchScalarGridSpec(
        num_scalar_prefetch=1,
        grid=(_NBLK,),
        in_specs=[
            pl.BlockSpec((_TB, _D), lambda b, info: (b, 0)),
            pl.BlockSpec((1, _D, _H), lambda b, info: (info[b], 0, 0)),
            pl.BlockSpec((1, 1, _H), lambda b, info: (info[b], 0, 0)),
            pl.BlockSpec((1, _H, _D), lambda b, info: (info[b], 0, 0)),
            pl.BlockSpec((1, 1, _D), lambda b, info: (info[b], 0, 0)),
        ],
        out_specs=pl.BlockSpec((_TB, _D), lambda b, info: (b, 0)),
    )
    ys = pl.pallas_call(
        _ffn_kernel,
        grid_spec=grid_spec,
        out_shape=jax.ShapeDtypeStruct((_CAP, _D), jnp.float32),
        compiler_params=pltpu.CompilerParams(
            dimension_semantics=("arbitrary",)),
    )(binfo, xs, w1b, b1r, w2b, b2r)

    gather = pl.kernel(
        _gather_kernel,
        mesh=mesh,
        out_type=[
            jax.ShapeDtypeStruct((_N, _D), jnp.float32),
            jax.ShapeDtypeStruct((_N, _D), jnp.float32),
        ],
        scratch_types=[
            pltpu.VMEM((_TPW,), jnp.int32),
            pltpu.VMEM((_TPW,), jnp.int32),
            pltpu.VMEM((32,), jnp.int32),
            pltpu.VMEM((32, _D), jnp.float32),
            pltpu.SemaphoreType.DMA,
        ],
    )
    y0, y1 = gather(d0.reshape(-1), d1.reshape(-1), ys)

    out = pl.pallas_call(
        _mix_kernel,
        out_shape=jax.ShapeDtypeStruct((_N, _D), jnp.float32),
    )(y0, y1, w1c, w2c)

    return out.reshape(x.shape), loss.reshape(())


# speedup vs baseline: 3.8220x; 1.2987x over previous
import jax
import jax.numpy as jnp
from jax import lax
from jax.experimental import pallas as pl
from jax.experimental.pallas import tpu as pltpu
from jax.experimental.pallas import tpu_sc as plsc

_N, _D, _E, _H = 2048, 768, 8, 3072
_K = 2
_P = _N * _K
_TB = 256
_CAP = _P + _E * _TB
_NBLK = _CAP // _TB
_NW = 32
_PPW = _P // _NW
_TPW = _N // _NW


def _gate_kernel(x_ref, wg_ref, d0_ref, d1_ref, w1_ref, w2_ref, binfo_ref,
                 loss_ref):
    x = x_ref[...]
    wg = wg_ref[...]
    logits = jax.lax.dot_general(
        x, wg, (((1,), (0,)), ((), ())),
        preferred_element_type=jnp.float32)
    iota = jax.lax.broadcasted_iota(jnp.int32, (_N, _E), 1)
    m1 = jnp.max(logits, axis=1, keepdims=True)
    i1 = jnp.min(jnp.where(logits == m1, iota, _E), axis=1, keepdims=True)
    masked = jnp.where(iota == i1, -jnp.inf, logits)
    m2 = jnp.max(masked, axis=1, keepdims=True)
    i2 = jnp.min(jnp.where(masked == m2, iota, _E), axis=1, keepdims=True)
    ew = jnp.exp(m2 - m1)
    denom = 1.0 + ew
    w1_ref[...] = 1.0 / denom
    w2_ref[...] = ew / denom

    gates = (jnp.where(iota == i1, 1.0 / denom, 0.0)
             + jnp.where(iota == i2, ew / denom, 0.0))
    load = jnp.sum((gates > 0.0).astype(jnp.float32), axis=0)
    mean = jnp.mean(load)
    var = jnp.sum((load - mean) ** 2) / (_E - 1)
    loss_ref[...] = jnp.full((1, 1), var / (mean * mean + 1e-6), jnp.float32)

    mask1 = jnp.where(iota == i1, 1.0, 0.0)
    mask2 = jnp.where(iota == i2, 1.0, 0.0)
    m12 = jnp.concatenate([mask1, mask2], axis=1)
    ir = jax.lax.broadcasted_iota(jnp.int32, (_N, _N), 0)
    ic = jax.lax.broadcasted_iota(jnp.int32, (_N, _N), 1)
    tri = jnp.where(ir >= ic, 1.0, 0.0)
    cs = jax.lax.dot_general(
        tri, m12, (((1,), (0,)), ((), ())),
        preferred_element_type=jnp.float32)
    cs1 = cs[:, 0:_E]
    cs2 = cs[:, _E:2 * _E]
    tot1 = cs1[_N - 1:_N, :]
    tot2 = cs2[_N - 1:_N, :]
    counts = tot1 + tot2
    padded = jnp.ceil(counts / _TB) * _TB
    ie_r = jax.lax.broadcasted_iota(jnp.int32, (_E, _E), 0)
    ie_c = jax.lax.broadcasted_iota(jnp.int32, (_E, _E), 1)
    padb = jnp.broadcast_to(padded, (_E, _E))
    offp = jnp.sum(jnp.where(ie_c < ie_r, padb, 0.0), axis=1,
                   keepdims=False).reshape(1, _E)

    rank1 = jnp.sum(jnp.where(iota == i1, cs1, 0.0), axis=1,
                    keepdims=True) - 1.0
    rank2 = (jnp.sum(jnp.where(iota == i2, cs2 + tot1, 0.0), axis=1,
                     keepdims=True) - 1.0)
    base1 = jnp.sum(jnp.where(iota == i1, offp, 0.0), axis=1, keepdims=True)
    base2 = jnp.sum(jnp.where(iota == i2, offp, 0.0), axis=1, keepdims=True)
    d0_ref[...] = (base1 + rank1).astype(jnp.int32)
    d1_ref[...] = (base2 + rank2).astype(jnp.int32)

    ib = jax.lax.broadcasted_iota(jnp.int32, (32, _E), 0)
    offp32 = jnp.broadcast_to(offp, (32, _E))
    bexp = jnp.sum(
        jnp.where((ib * _TB).astype(jnp.float32) >= offp32, 1.0, 0.0),
        axis=1) - 1.0
    nblk = jnp.sum(padded) / _TB
    i32v = jax.lax.broadcasted_iota(jnp.int32, (32,), 0)
    binfo_ref[...] = jnp.where(i32v == 31, nblk, bexp).astype(jnp.int32)


def _dispatch_kernel(dest_hbm, x_hbm, xs_hbm, dest_v, idx_v, rows_v, sem):
    c = lax.axis_index("c")
    s = lax.axis_index("s")
    wid = s * 2 + c
    my_start = wid * _PPW
    t0 = my_start - (my_start // _N) * _N
    pltpu.sync_copy(dest_hbm.at[pl.ds(my_start, _PPW)], dest_v)
    for cc in range(4):
        pltpu.sync_copy(x_hbm.at[pl.ds(t0 + cc * 32, 32)], rows_v)
        for q in range(2):
            idx_v[(q * 16):(q * 16 + 16)] = \
                dest_v[(cc * 32 + q * 16):(cc * 32 + q * 16 + 16)]
        pltpu.async_copy(rows_v, xs_hbm.at[idx_v], sem).wait()


def _ffn_kernel(binfo_ref, xs_ref, w1_ref, b1_ref, w2_ref, b2_ref, ys_ref):
    b = pl.program_id(0)
    nb = binfo_ref[31]

    @pl.when(b < nb)
    def _():
        xc = xs_ref[...].astype(jnp.bfloat16)
        h = jnp.dot(xc, w1_ref[0].astype(jnp.bfloat16),
                    preferred_element_type=jnp.float32)
        h = h + b1_ref[0]
        h = 0.5 * h * (1.0 + jax.lax.erf(h * 0.7071067811865476))
        y = jnp.dot(h.astype(jnp.bfloat16), w2_ref[0].astype(jnp.bfloat16),
                    preferred_element_type=jnp.float32)
        ys_ref[...] = y + b2_ref[0]


def _gather_kernel(d0_hbm, d1_hbm, ys_hbm, y0_hbm, y1_hbm,
                   d0_v, d1_v, idx_v, rows_v, sem):
    c = lax.axis_index("c")
    s = lax.axis_index("s")
    wid = s * 2 + c
    tbase = wid * _TPW
    pltpu.sync_copy(d0_hbm.at[pl.ds(tbase, _TPW)], d0_v)
    pltpu.sync_copy(d1_hbm.at[pl.ds(tbase, _TPW)], d1_v)
    for cc in range(_TPW // 32):
        for q in range(2):
            idx_v[(q * 16):(q * 16 + 16)] = \
                d0_v[(cc * 32 + q * 16):(cc * 32 + q * 16 + 16)]
        pltpu.async_copy(ys_hbm.at[idx_v], rows_v, sem).wait()
        pltpu.sync_copy(rows_v, y0_hbm.at[pl.ds(tbase + cc * 32, 32)])
        for q in range(2):
            idx_v[(q * 16):(q * 16 + 16)] = \
                d1_v[(cc * 32 + q * 16):(cc * 32 + q * 16 + 16)]
        pltpu.async_copy(ys_hbm.at[idx_v], rows_v, sem).wait()
        pltpu.sync_copy(rows_v, y1_hbm.at[pl.ds(tbase + cc * 32, 32)])


def _mix_kernel(y0_ref, y1_ref, w1_ref, w2_ref, out_ref):
    out_ref[...] = y0_ref[...] * w1_ref[...] + y1_ref[...] * w2_ref[...]


def kernel(x, w_gate, fc1_w, fc1_b, fc2_w, fc2_b):
    x2 = x.reshape(_N, _D)
    d0, d1, w1c, w2c, binfo, loss = pl.pallas_call(
        _gate_kernel,
        out_shape=(
            jax.ShapeDtypeStruct((_N, 1), jnp.int32),
            jax.ShapeDtypeStruct((_N, 1), jnp.int32),
            jax.ShapeDtypeStruct((_N, 1), jnp.float32),
            jax.ShapeDtypeStruct((_N, 1), jnp.float32),
            jax.ShapeDtypeStruct((32,), jnp.int32),
            jax.ShapeDtypeStruct((1, 1), jnp.float32),
        ),
    )(x2, w_gate)

    dest = jnp.concatenate([d0.reshape(-1), d1.reshape(-1)])

    mesh = plsc.VectorSubcoreMesh(core_axis_name="c", subcore_axis_name="s")

    dispatch = pl.kernel(
        _dispatch_kernel,
        mesh=mesh,
        out_type=jax.ShapeDtypeStruct((_CAP, _D), jnp.float32),
        scratch_types=[
            pltpu.VMEM((_PPW,), jnp.int32),
            pltpu.VMEM((32,), jnp.int32),
            pltpu.VMEM((32, _D), jnp.float32),
            pltpu.SemaphoreType.DMA,
        ],
    )
    xs = dispatch(dest, x2)

    b1r = fc1_b.reshape(_E, 1, _H)
    b2r = fc2_b.reshape(_E, 1, _D)
    grid_spec = pltpu.PrefetchScalarGridSpec(
        num_scalar_prefetch=1,
        grid=(_NBLK,),
        in_specs=[
            pl.BlockSpec((_TB, _D), lambda b, info: (b, 0)),
            pl.BlockSpec((1, _D, _H), lambda b, info: (info[b], 0, 0)),
            pl.BlockSpec((1, 1, _H), lambda b, info: (info[b], 0, 0)),
            pl.BlockSpec((1, _H, _D), lambda b, info: (info[b], 0, 0)),
            pl.BlockSpec((1, 1, _D), lambda b, info: (info[b], 0, 0)),
        ],
        out_specs=pl.BlockSpec((_TB, _D), lambda b, info: (b, 0)),
    )
    ys = pl.pallas_call(
        _ffn_kernel,
        grid_spec=grid_spec,
        out_shape=jax.ShapeDtypeStruct((_CAP, _D), jnp.float32),
        compiler_params=pltpu.CompilerParams(
            dimension_semantics=("arbitrary",)),
    )(binfo, xs, fc1_w, b1r, fc2_w, b2r)

    gather = pl.kernel(
        _gather_kernel,
        mesh=mesh,
        out_type=[
            jax.ShapeDtypeStruct((_N, _D), jnp.float32),
            jax.ShapeDtypeStruct((_N, _D), jnp.float32),
        ],
        scratch_types=[
            pltpu.VMEM((_TPW,), jnp.int32),
            pltpu.VMEM((_TPW,), jnp.int32),
            pltpu.VMEM((32,), jnp.int32),
            pltpu.VMEM((32, _D), jnp.float32),
            pltpu.SemaphoreType.DMA,
        ],
    )
    y0, y1 = gather(d0.reshape(-1), d1.reshape(-1), ys)

    out = pl.pallas_call(
        _mix_kernel,
        out_shape=jax.ShapeDtypeStruct((_N, _D), jnp.float32),
    )(y0, y1, w1c, w2c)

    return out.reshape(x.shape), loss.reshape(())


# speedup vs baseline: 4.0595x; 1.0621x over previous
import jax
import jax.numpy as jnp
from jax import lax
from jax.experimental import pallas as pl
from jax.experimental.pallas import tpu as pltpu
from jax.experimental.pallas import tpu_sc as plsc

_N, _D, _E, _H = 2048, 768, 8, 3072
_K = 2
_P = _N * _K
_TB = 256
_CAP = _P + _E * _TB
_NBLK = _CAP // _TB
_NW = 32
_PPW = _P // _NW
_TPW = _N // _NW


def _gate_kernel(x_ref, wg_ref, d0_ref, d1_ref, w1_ref, w2_ref, binfo_ref,
                 loss_ref):
    x = x_ref[...]
    wg = wg_ref[...]
    logits = jax.lax.dot_general(
        x, wg, (((1,), (0,)), ((), ())),
        preferred_element_type=jnp.float32)
    iota = jax.lax.broadcasted_iota(jnp.int32, (_N, _E), 1)
    m1 = jnp.max(logits, axis=1, keepdims=True)
    i1 = jnp.min(jnp.where(logits == m1, iota, _E), axis=1, keepdims=True)
    masked = jnp.where(iota == i1, -jnp.inf, logits)
    m2 = jnp.max(masked, axis=1, keepdims=True)
    i2 = jnp.min(jnp.where(masked == m2, iota, _E), axis=1, keepdims=True)
    ew = jnp.exp(m2 - m1)
    denom = 1.0 + ew
    w1_ref[...] = jnp.broadcast_to(1.0 / denom, (_N, 16))
    w2_ref[...] = jnp.broadcast_to(ew / denom, (_N, 16))

    gates = (jnp.where(iota == i1, 1.0 / denom, 0.0)
             + jnp.where(iota == i2, ew / denom, 0.0))
    load = jnp.sum((gates > 0.0).astype(jnp.float32), axis=0)
    mean = jnp.mean(load)
    var = jnp.sum((load - mean) ** 2) / (_E - 1)
    loss_ref[...] = jnp.full((1, 1), var / (mean * mean + 1e-6), jnp.float32)

    mask1 = jnp.where(iota == i1, 1.0, 0.0)
    mask2 = jnp.where(iota == i2, 1.0, 0.0)
    m12 = jnp.concatenate([mask1, mask2], axis=1)
    ir = jax.lax.broadcasted_iota(jnp.int32, (_N, _N), 0)
    ic = jax.lax.broadcasted_iota(jnp.int32, (_N, _N), 1)
    tri = jnp.where(ir >= ic, 1.0, 0.0)
    cs = jax.lax.dot_general(
        tri, m12, (((1,), (0,)), ((), ())),
        preferred_element_type=jnp.float32)
    cs1 = cs[:, 0:_E]
    cs2 = cs[:, _E:2 * _E]
    tot1 = cs1[_N - 1:_N, :]
    tot2 = cs2[_N - 1:_N, :]
    counts = tot1 + tot2
    padded = jnp.ceil(counts / _TB) * _TB
    ie_r = jax.lax.broadcasted_iota(jnp.int32, (_E, _E), 0)
    ie_c = jax.lax.broadcasted_iota(jnp.int32, (_E, _E), 1)
    padb = jnp.broadcast_to(padded, (_E, _E))
    offp = jnp.sum(jnp.where(ie_c < ie_r, padb, 0.0), axis=1,
                   keepdims=False).reshape(1, _E)

    rank1 = jnp.sum(jnp.where(iota == i1, cs1, 0.0), axis=1,
                    keepdims=True) - 1.0
    rank2 = (jnp.sum(jnp.where(iota == i2, cs2 + tot1, 0.0), axis=1,
                     keepdims=True) - 1.0)
    base1 = jnp.sum(jnp.where(iota == i1, offp, 0.0), axis=1, keepdims=True)
    base2 = jnp.sum(jnp.where(iota == i2, offp, 0.0), axis=1, keepdims=True)
    d0_ref[...] = (base1 + rank1).astype(jnp.int32)
    d1_ref[...] = (base2 + rank2).astype(jnp.int32)

    ib = jax.lax.broadcasted_iota(jnp.int32, (32, _E), 0)
    offp32 = jnp.broadcast_to(offp, (32, _E))
    bexp = jnp.sum(
        jnp.where((ib * _TB).astype(jnp.float32) >= offp32, 1.0, 0.0),
        axis=1) - 1.0
    nblk = jnp.sum(padded) / _TB
    i32v = jax.lax.broadcasted_iota(jnp.int32, (32,), 0)
    binfo_ref[...] = jnp.where(i32v == 31, nblk, bexp).astype(jnp.int32)


def _dispatch_kernel(dest_hbm, x_hbm, xs_hbm, dest_v, rows_v, sem):
    c = lax.axis_index("c")
    s = lax.axis_index("s")
    wid = s * 2 + c
    my_start = wid * _PPW
    t0 = my_start - (my_start // _N) * _N
    pltpu.sync_copy(dest_hbm.at[pl.ds(my_start, _PPW)], dest_v)
    pltpu.sync_copy(x_hbm.at[pl.ds(t0, _PPW)], rows_v)
    pltpu.async_copy(rows_v, xs_hbm.at[dest_v], sem).wait()


def _ffn_kernel(binfo_ref, xs_ref, w1_ref, b1_ref, w2_ref, b2_ref, ys_ref):
    b = pl.program_id(0)
    nb = binfo_ref[31]

    @pl.when(b < nb)
    def _():
        xc = xs_ref[...].astype(jnp.bfloat16)
        h = jnp.dot(xc, w1_ref[0].astype(jnp.bfloat16),
                    preferred_element_type=jnp.float32)
        h = h + b1_ref[0]
        h = 0.5 * h * (1.0 + jax.lax.erf(h * 0.7071067811865476))
        y = jnp.dot(h.astype(jnp.bfloat16), w2_ref[0].astype(jnp.bfloat16),
                    preferred_element_type=jnp.float32)
        ys_ref[...] = y + b2_ref[0]


def _combine_kernel(d0_hbm, d1_hbm, w1b_hbm, w2b_hbm, ys_hbm, out_hbm,
                    d0_v, d1_v, w0_v, w1_v, y0_v, y1_v, sem):
    c = lax.axis_index("c")
    s = lax.axis_index("s")
    wid = s * 2 + c
    tbase = wid * _TPW
    pltpu.sync_copy(d0_hbm.at[pl.ds(tbase, _TPW)], d0_v)
    pltpu.sync_copy(d1_hbm.at[pl.ds(tbase, _TPW)], d1_v)
    pltpu.sync_copy(w1b_hbm.at[pl.ds(tbase, _TPW)], w0_v)
    pltpu.sync_copy(w2b_hbm.at[pl.ds(tbase, _TPW)], w1_v)
    h0 = pltpu.async_copy(ys_hbm.at[d0_v], y0_v, sem)
    h1 = pltpu.async_copy(ys_hbm.at[d1_v], y1_v, sem)
    h0.wait()
    h1.wait()

    def row_body(r, carry):
        wv0 = w0_v[r]
        wv1 = w1_v[r]
        for col in range(_D // 16):
            a = y0_v[r, pl.ds(col * 16, 16)]
            b = y1_v[r, pl.ds(col * 16, 16)]
            y0_v[r, pl.ds(col * 16, 16)] = a * wv0 + b * wv1
        return carry

    lax.fori_loop(0, _TPW, row_body, 0)
    pltpu.sync_copy(y0_v, out_hbm.at[pl.ds(tbase, _TPW)])


def kernel(x, w_gate, fc1_w, fc1_b, fc2_w, fc2_b):
    x2 = x.reshape(_N, _D)
    d0, d1, w1c, w2c, binfo, loss = pl.pallas_call(
        _gate_kernel,
        out_shape=(
            jax.ShapeDtypeStruct((_N, 1), jnp.int32),
            jax.ShapeDtypeStruct((_N, 1), jnp.int32),
            jax.ShapeDtypeStruct((_N, 16), jnp.float32),
            jax.ShapeDtypeStruct((_N, 16), jnp.float32),
            jax.ShapeDtypeStruct((32,), jnp.int32),
            jax.ShapeDtypeStruct((1, 1), jnp.float32),
        ),
    )(x2, w_gate)

    dest = jnp.concatenate([d0.reshape(-1), d1.reshape(-1)])

    mesh = plsc.VectorSubcoreMesh(core_axis_name="c", subcore_axis_name="s")

    dispatch = pl.kernel(
        _dispatch_kernel,
        mesh=mesh,
        out_type=jax.ShapeDtypeStruct((_CAP, _D), jnp.float32),
        scratch_types=[
            pltpu.VMEM((_PPW,), jnp.int32),
            pltpu.VMEM((_PPW, _D), jnp.float32),
            pltpu.SemaphoreType.DMA,
        ],
    )
    xs = dispatch(dest, x2)

    b1r = fc1_b.reshape(_E, 1, _H)
    b2r = fc2_b.reshape(_E, 1, _D)
    grid_spec = pltpu.PrefetchScalarGridSpec(
        num_scalar_prefetch=1,
        grid=(_NBLK,),
        in_specs=[
            pl.BlockSpec((_TB, _D), lambda b, info: (b, 0)),
            pl.BlockSpec((1, _D, _H), lambda b, info: (info[b], 0, 0)),
            pl.BlockSpec((1, 1, _H), lambda b, info: (info[b], 0, 0)),
            pl.BlockSpec((1, _H, _D), lambda b, info: (info[b], 0, 0)),
            pl.BlockSpec((1, 1, _D), lambda b, info: (info[b], 0, 0)),
        ],
        out_specs=pl.BlockSpec((_TB, _D), lambda b, info: (b, 0)),
    )
    ys = pl.pallas_call(
        _ffn_kernel,
        grid_spec=grid_spec,
        out_shape=jax.ShapeDtypeStruct((_CAP, _D), jnp.float32),
        compiler_params=pltpu.CompilerParams(
            dimension_semantics=("arbitrary",)),
    )(binfo, xs, fc1_w, b1r, fc2_w, b2r)

    combine = pl.kernel(
        _combine_kernel,
        mesh=mesh,
        out_type=jax.ShapeDtypeStruct((_N, _D), jnp.float32),
        scratch_types=[
            pltpu.VMEM((_TPW,), jnp.int32),
            pltpu.VMEM((_TPW,), jnp.int32),
            pltpu.VMEM((_TPW, 16), jnp.float32),
            pltpu.VMEM((_TPW, 16), jnp.float32),
            pltpu.VMEM((_TPW, _D), jnp.float32),
            pltpu.VMEM((_TPW, _D), jnp.float32),
            pltpu.SemaphoreType.DMA,
        ],
    )
    out = combine(d0.reshape(-1), d1.reshape(-1), w1c, w2c, ys)

    return out.reshape(x.shape), loss.reshape(())
